# baseline (speedup 1.0000x reference)
import functools

import jax
import jax.numpy as jnp
from jax import lax
from jax.experimental import pallas as pl
from jax.experimental.pallas import tpu as pltpu
from jax.experimental.pallas import tpu_sc as plsc

_N = 10000
_E = 320000
_D = 128
_NC = 2
_NS = 16
_NW = _NC * _NS
_C = 80
_NCHUNK = 128
_PER_W = _NCHUNK * _C
_EPAD = _NW * _PER_W
_NBUF = 2
_G = 8
_NGRP = _NCHUNK // _G
_NSINK = 8
_NDEG = 10240
_RPT = 624
_TAIL = _N - _NS * _RPT
_TAIL_OFF = _NS * _RPT


def _make_seg_sum(with_deg):
  mesh = plsc.VectorSubcoreMesh(core_axis_name="c", subcore_axis_name="s")
  out_type = [jax.ShapeDtypeStruct((_NC, _N, _D), jnp.float32)]
  if with_deg:
    out_type.append(jax.ShapeDtypeStruct((_NC, _NDEG), jnp.float32))

  @functools.partial(
      pl.kernel,
      out_type=out_type,
      mesh=mesh,
      scratch_types=[
          pltpu.VMEM((_G, _C), jnp.int32),
          pltpu.VMEM((_G, _C), jnp.int32),
          pltpu.VMEM((_NBUF, _C, _D), jnp.float32),
          pltpu.VMEM((_C,), jnp.float32),
          pltpu.VMEM_SHARED((_N + _NSINK, _D), jnp.float32),
          pltpu.VMEM_SHARED((_NDEG,), jnp.float32),
          pltpu.SemaphoreType.DMA,
          pltpu.SemaphoreType.DMA,
          pltpu.SemaphoreType.DMA,
          pltpu.SemaphoreType.DMA,
      ],
  )
  def seg(table_hbm, src_hbm, dst_hbm, ones_hbm, zrow_hbm, zdeg_hbm,
          *out_and_scratch):
    if with_deg:
      (agg_out, deg_out, srcblk, dstblk, rows_v, ones_v, acc_sh, deg_sh,
       sem0, sem1, sem2, sem3) = out_and_scratch
    else:
      (agg_out, srcblk, dstblk, rows_v, ones_v, acc_sh, deg_sh,
       sem0, sem1, sem2, sem3) = out_and_scratch
      deg_out = None
    sems = (sem0, sem1, sem2, sem3)
    c = lax.axis_index("c")
    s = lax.axis_index("s")
    wid = s * _NC + c
    pltpu.sync_copy(zrow_hbm.at[pl.ds(s * _RPT, _RPT)],
                    acc_sh.at[pl.ds(s * _RPT, _RPT)])

    @pl.when(s == _NS - 1)
    def _():
      pltpu.sync_copy(zrow_hbm.at[pl.ds(_TAIL_OFF, _TAIL)],
                      acc_sh.at[pl.ds(_TAIL_OFF, _TAIL)])

    if with_deg:
      @pl.when(s == 0)
      def _():
        pltpu.sync_copy(zdeg_hbm, deg_sh)
      pltpu.sync_copy(ones_hbm, ones_v)
    plsc.subcore_barrier()

    def body(g, carry):
      g8 = pl.multiple_of(g * _G, _G)
      pltpu.sync_copy(src_hbm.at[wid, pl.ds(g8, _G)], srcblk)
      pltpu.sync_copy(dst_hbm.at[wid, pl.ds(g8, _G)], dstblk)
      for half in range(_G // _NBUF):
        hs = []
        for k in range(_NBUF):
          j = half * _NBUF + k
          hs.append(pltpu.async_copy(
              table_hbm.at[srcblk.at[j]], rows_v.at[k], sems[k]))
        for k in range(_NBUF):
          j = half * _NBUF + k
          hs[k].wait()
          pltpu.sync_copy(rows_v.at[k], acc_sh.at[dstblk.at[j]], add=True)
          if with_deg:
            pltpu.sync_copy(ones_v, deg_sh.at[dstblk.at[j]], add=True)
      return carry

    lax.fori_loop(0, _NGRP, body, 0)
    plsc.subcore_barrier()

    pltpu.sync_copy(acc_sh.at[pl.ds(s * _RPT, _RPT)],
                    agg_out.at[c, pl.ds(s * _RPT, _RPT)])

    @pl.when(s == _NS - 1)
    def _():
      pltpu.sync_copy(acc_sh.at[pl.ds(_TAIL_OFF, _TAIL)],
                      agg_out.at[c, pl.ds(_TAIL_OFF, _TAIL)])

    if with_deg:
      @pl.when(s == 0)
      def _():
        pltpu.sync_copy(deg_sh, deg_out.at[c])

  return seg


def _prelu(v, a):
  return jnp.where(v >= 0, v, a * v)


_BLK = 1000


def _tc1_body(aggp, degp, x, w0l, b0l, w0r, wskip, a0, hin2, degc):
  agg = aggp[0] + aggp[1]
  deg = jnp.maximum(degp[0] + degp[1], 1.0)
  mean = agg / deg
  h = (jnp.dot(mean, w0l[...], preferred_element_type=jnp.float32)
       + b0l[...]
       + jnp.dot(x[...], w0r[...], preferred_element_type=jnp.float32))
  h = _prelu(_prelu(h, a0[...]), a0[...])
  hin2[...] = h + jnp.dot(x[...], wskip[...], preferred_element_type=jnp.float32)
  degc[...] = deg


def _tc2_body(aggp, degc, hin2, w1l, b1l, w1r, a1, out):
  mean = (aggp[0] + aggp[1]) / degc[...]
  h = (jnp.dot(mean, w1l[...], preferred_element_type=jnp.float32)
       + b1l[...]
       + jnp.dot(hin2[...], w1r[...], preferred_element_type=jnp.float32))
  out[...] = _prelu(h, a1[...])


def _w_spec():
  return pl.BlockSpec((_D, _D), lambda i: (0, 0))


def _v_spec():
  return pl.BlockSpec((1, _D), lambda i: (0, 0))


def _row_spec(d=_D):
  return pl.BlockSpec((_BLK, d), lambda i: (i, 0))


def _part_spec(d=_D):
  return pl.BlockSpec((_NC, _BLK, d), lambda i: (0, i, 0))


def _tc1(aggp, degp, x, w0l, b0l, w0r, wskip, a0):
  return pl.pallas_call(
      _tc1_body,
      grid=(_N // _BLK,),
      in_specs=[_part_spec(), _part_spec(1), _row_spec(), _w_spec(), _v_spec(),
                _w_spec(), _w_spec(), _v_spec()],
      out_specs=[_row_spec(), _row_spec(1)],
      out_shape=[jax.ShapeDtypeStruct((_N, _D), jnp.float32),
                 jax.ShapeDtypeStruct((_N, 1), jnp.float32)],
  )(aggp, degp, x, w0l, b0l, w0r, wskip, a0)


def _tc2(aggp, degc, hin2, w1l, b1l, w1r, a1):
  return pl.pallas_call(
      _tc2_body,
      grid=(_N // _BLK,),
      in_specs=[_part_spec(), _row_spec(1), _row_spec(), _w_spec(), _v_spec(),
                _w_spec(), _v_spec()],
      out_specs=_row_spec(),
      out_shape=jax.ShapeDtypeStruct((_N, _D), jnp.float32),
  )(aggp, degc, hin2, w1l, b1l, w1r, a1)


def kernel(x, edge_index, edge_weight, W0l, b0l, W0r, W1l, b1l, W1r, Wskip, a0, a1):
  del edge_weight
  npad = _EPAD - _E
  src = jnp.concatenate(
      [edge_index[0].astype(jnp.int32), jnp.zeros((npad,), jnp.int32)])
  dst = jnp.concatenate(
      [edge_index[1].astype(jnp.int32), jnp.full((npad,), _N, jnp.int32)])
  src3 = src.reshape(_NW, _NCHUNK, _C)
  dst3 = dst.reshape(_NW, _NCHUNK, _C)
  ones = jnp.ones((_C,), jnp.float32)
  zrow = jnp.zeros((_N, _D), jnp.float32)
  zdeg = jnp.zeros((_NDEG,), jnp.float32)
  seg1 = _make_seg_sum(True)
  seg2 = _make_seg_sum(False)

  b0 = b0l.reshape(1, _D)
  b1 = b1l.reshape(1, _D)
  a0r = a0.reshape(1, _D)
  a1r = a1.reshape(1, _D)

  agg1p, deg1p = seg1(x, src3, dst3, ones, zrow, zdeg)
  deg1p = deg1p[:, :_N].reshape(_NC, _N, 1)
  hin2, degc = _tc1(agg1p, deg1p, x, W0l, b0, W0r, Wskip, a0r)
  (agg2p,) = seg2(hin2, src3, dst3, ones, zrow, zdeg)
  return _tc2(agg2p, degc, hin2, W1l, b1, W1r, a1r)


# speedup vs baseline: 1.0157x; 1.0157x over previous
import functools

import jax
import jax.numpy as jnp
from jax import lax
from jax.experimental import pallas as pl
from jax.experimental.pallas import tpu as pltpu
from jax.experimental.pallas import tpu_sc as plsc

_N = 10000
_E = 320000
_D = 128
_NC = 2
_NS = 16
_NW = _NC * _NS
_C = 80
_NCHUNK = 128
_PER_W = _NCHUNK * _C
_EPAD = _NW * _PER_W
_NBUF = 4
_G = 8
_NGRP = _NCHUNK // _G
_NSINK = 128
_NDEG = 10240
_RPT = 624
_TAIL = _N - _NS * _RPT
_TAIL_OFF = _NS * _RPT


def _make_seg_sum(with_deg):
  mesh = plsc.VectorSubcoreMesh(core_axis_name="c", subcore_axis_name="s")
  out_type = [jax.ShapeDtypeStruct((_NC, _N, _D), jnp.float32)]
  if with_deg:
    out_type.append(jax.ShapeDtypeStruct((_NC, _NDEG), jnp.float32))

  @functools.partial(
      pl.kernel,
      out_type=out_type,
      mesh=mesh,
      scratch_types=[
          pltpu.VMEM((_G, _C), jnp.int32),
          pltpu.VMEM((_G, _C), jnp.int32),
          pltpu.VMEM((_NBUF, _C, _D), jnp.float32),
          pltpu.VMEM((_C,), jnp.float32),
          pltpu.VMEM_SHARED((_N + _NSINK, _D), jnp.float32),
          pltpu.VMEM_SHARED((_NDEG,), jnp.float32),
          pltpu.SemaphoreType.DMA,
          pltpu.SemaphoreType.DMA,
          pltpu.SemaphoreType.DMA,
          pltpu.SemaphoreType.DMA,
      ],
  )
  def seg(table_hbm, src_hbm, dst_hbm, ones_hbm, zrow_hbm, zdeg_hbm,
          *out_and_scratch):
    if with_deg:
      (agg_out, deg_out, srcblk, dstblk, rows_v, ones_v, acc_sh, deg_sh,
       sem0, sem1, sem2, sem3) = out_and_scratch
    else:
      (agg_out, srcblk, dstblk, rows_v, ones_v, acc_sh, deg_sh,
       sem0, sem1, sem2, sem3) = out_and_scratch
      deg_out = None
    sems = (sem0, sem1, sem2, sem3)
    c = lax.axis_index("c")
    s = lax.axis_index("s")
    wid = s * _NC + c
    pltpu.sync_copy(zrow_hbm.at[pl.ds(s * _RPT, _RPT)],
                    acc_sh.at[pl.ds(s * _RPT, _RPT)])

    @pl.when(s == _NS - 1)
    def _():
      pltpu.sync_copy(zrow_hbm.at[pl.ds(_TAIL_OFF, _TAIL)],
                      acc_sh.at[pl.ds(_TAIL_OFF, _TAIL)])

    if with_deg:
      @pl.when(s == 0)
      def _():
        pltpu.sync_copy(zdeg_hbm, deg_sh)
      pltpu.sync_copy(ones_hbm, ones_v)
    plsc.subcore_barrier()

    def body(g, carry):
      g8 = pl.multiple_of(g * _G, _G)
      pltpu.sync_copy(src_hbm.at[wid, pl.ds(g8, _G)], srcblk)
      pltpu.sync_copy(dst_hbm.at[wid, pl.ds(g8, _G)], dstblk)
      for half in range(_G // _NBUF):
        hs = []
        for k in range(_NBUF):
          j = half * _NBUF + k
          hs.append(pltpu.async_copy(
              table_hbm.at[srcblk.at[j]], rows_v.at[k], sems[k]))
        for k in range(_NBUF):
          j = half * _NBUF + k
          hs[k].wait()
          pltpu.sync_copy(rows_v.at[k], acc_sh.at[dstblk.at[j]], add=True)
          if with_deg:
            pltpu.sync_copy(ones_v, deg_sh.at[dstblk.at[j]], add=True)
      return carry

    lax.fori_loop(0, _NGRP, body, 0)
    plsc.subcore_barrier()

    pltpu.sync_copy(acc_sh.at[pl.ds(s * _RPT, _RPT)],
                    agg_out.at[c, pl.ds(s * _RPT, _RPT)])

    @pl.when(s == _NS - 1)
    def _():
      pltpu.sync_copy(acc_sh.at[pl.ds(_TAIL_OFF, _TAIL)],
                      agg_out.at[c, pl.ds(_TAIL_OFF, _TAIL)])

    if with_deg:
      @pl.when(s == 0)
      def _():
        pltpu.sync_copy(deg_sh, deg_out.at[c])

  return seg


def _prelu(v, a):
  return jnp.where(v >= 0, v, a * v)


_BLK = 1000


def _tc1_body(aggp, degp, x, w0l, b0l, w0r, wskip, a0, hin2, degc):
  agg = aggp[0] + aggp[1]
  deg = jnp.maximum(degp[0] + degp[1], 1.0)
  mean = agg / deg
  h = (jnp.dot(mean, w0l[...], preferred_element_type=jnp.float32)
       + b0l[...]
       + jnp.dot(x[...], w0r[...], preferred_element_type=jnp.float32))
  h = _prelu(_prelu(h, a0[...]), a0[...])
  hin2[...] = h + jnp.dot(x[...], wskip[...], preferred_element_type=jnp.float32)
  degc[...] = deg


def _tc2_body(aggp, degc, hin2, w1l, b1l, w1r, a1, out):
  mean = (aggp[0] + aggp[1]) / degc[...]
  h = (jnp.dot(mean, w1l[...], preferred_element_type=jnp.float32)
       + b1l[...]
       + jnp.dot(hin2[...], w1r[...], preferred_element_type=jnp.float32))
  out[...] = _prelu(h, a1[...])


def _w_spec():
  return pl.BlockSpec((_D, _D), lambda i: (0, 0))


def _v_spec():
  return pl.BlockSpec((1, _D), lambda i: (0, 0))


def _row_spec(d=_D):
  return pl.BlockSpec((_BLK, d), lambda i: (i, 0))


def _part_spec(d=_D):
  return pl.BlockSpec((_NC, _BLK, d), lambda i: (0, i, 0))


def _tc1(aggp, degp, x, w0l, b0l, w0r, wskip, a0):
  return pl.pallas_call(
      _tc1_body,
      grid=(_N // _BLK,),
      in_specs=[_part_spec(), _part_spec(1), _row_spec(), _w_spec(), _v_spec(),
                _w_spec(), _w_spec(), _v_spec()],
      out_specs=[_row_spec(), _row_spec(1)],
      out_shape=[jax.ShapeDtypeStruct((_N, _D), jnp.float32),
                 jax.ShapeDtypeStruct((_N, 1), jnp.float32)],
  )(aggp, degp, x, w0l, b0l, w0r, wskip, a0)


def _tc2(aggp, degc, hin2, w1l, b1l, w1r, a1):
  return pl.pallas_call(
      _tc2_body,
      grid=(_N // _BLK,),
      in_specs=[_part_spec(), _row_spec(1), _row_spec(), _w_spec(), _v_spec(),
                _w_spec(), _v_spec()],
      out_specs=_row_spec(),
      out_shape=jax.ShapeDtypeStruct((_N, _D), jnp.float32),
  )(aggp, degc, hin2, w1l, b1l, w1r, a1)


def kernel(x, edge_index, edge_weight, W0l, b0l, W0r, W1l, b1l, W1r, Wskip, a0, a1):
  del edge_weight
  npad = _EPAD - _E
  src = jnp.concatenate(
      [edge_index[0].astype(jnp.int32), jnp.zeros((npad,), jnp.int32)])
  dst = jnp.concatenate(
      [edge_index[1].astype(jnp.int32),
       _N + jnp.arange(npad, dtype=jnp.int32) % _NSINK])
  src3 = src.reshape(_NW, _NCHUNK, _C)
  dst3 = dst.reshape(_NW, _NCHUNK, _C)
  ones = jnp.ones((_C,), jnp.float32)
  zrow = jnp.zeros((_N, _D), jnp.float32)
  zdeg = jnp.zeros((_NDEG,), jnp.float32)
  seg1 = _make_seg_sum(True)
  seg2 = _make_seg_sum(False)

  b0 = b0l.reshape(1, _D)
  b1 = b1l.reshape(1, _D)
  a0r = a0.reshape(1, _D)
  a1r = a1.reshape(1, _D)

  agg1p, deg1p = seg1(x, src3, dst3, ones, zrow, zdeg)
  deg1p = deg1p[:, :_N].reshape(_NC, _N, 1)
  hin2, degc = _tc1(agg1p, deg1p, x, W0l, b0, W0r, Wskip, a0r)
  (agg2p,) = seg2(hin2, src3, dst3, ones, zrow, zdeg)
  return _tc2(agg2p, degc, hin2, W1l, b1, W1r, a1r)


# speedup vs baseline: 2.6840x; 2.6424x over previous
import functools

import jax
import jax.numpy as jnp
from jax import lax
from jax.experimental import pallas as pl
from jax.experimental.pallas import tpu as pltpu
from jax.experimental.pallas import tpu_sc as plsc

_N = 10000
_E = 320000
_D = 128
_NC = 2
_NS = 16
_NW = _NC * _NS
_C = 80
_NCHUNK = 128
_PER_W = _NCHUNK * _C
_EPAD = _NW * _PER_W
_NBUF = 4
_G = 8
_NGRP = _NCHUNK // _G
_NSINK = 128
_NDEG = 10240
_RPT = 624
_TAIL = _N - _NS * _RPT
_TAIL_OFF = _NS * _RPT


def _make_seg_sum(with_deg):
  mesh = plsc.VectorSubcoreMesh(core_axis_name="c", subcore_axis_name="s")
  out_type = [jax.ShapeDtypeStruct((_NC, _N, _D), jnp.float32)]
  if with_deg:
    out_type.append(jax.ShapeDtypeStruct((_NC, _NDEG), jnp.float32))

  @functools.partial(
      pl.kernel,
      out_type=out_type,
      mesh=mesh,
      scratch_types=[
          pltpu.VMEM((_G, _C), jnp.int32),
          pltpu.VMEM((_G, _C), jnp.int32),
          pltpu.VMEM((_NBUF, _C, _D), jnp.float32),
          pltpu.VMEM((_C,), jnp.float32),
          pltpu.VMEM_SHARED((_N + _NSINK, _D), jnp.float32),
          pltpu.VMEM_SHARED((_NDEG,), jnp.float32),
          pltpu.SemaphoreType.DMA,
          pltpu.SemaphoreType.DMA,
          pltpu.SemaphoreType.DMA,
          pltpu.SemaphoreType.DMA,
      ],
  )
  def seg(table_hbm, src_hbm, dst_hbm, ones_hbm, zrow_hbm, zdeg_hbm,
          *out_and_scratch):
    if with_deg:
      (agg_out, deg_out, srcblk, dstblk, rows_v, ones_v, acc_sh, deg_sh,
       sem0, sem1, sem2, sem3) = out_and_scratch
    else:
      (agg_out, srcblk, dstblk, rows_v, ones_v, acc_sh, deg_sh,
       sem0, sem1, sem2, sem3) = out_and_scratch
      deg_out = None
    sems = (sem0, sem1, sem2, sem3)
    c = lax.axis_index("c")
    s = lax.axis_index("s")
    wid = s * _NC + c
    pltpu.sync_copy(zrow_hbm.at[pl.ds(s * _RPT, _RPT)],
                    acc_sh.at[pl.ds(s * _RPT, _RPT)])

    @pl.when(s == _NS - 1)
    def _():
      pltpu.sync_copy(zrow_hbm.at[pl.ds(_TAIL_OFF, _TAIL)],
                      acc_sh.at[pl.ds(_TAIL_OFF, _TAIL)])

    if with_deg:
      @pl.when(s == 0)
      def _():
        pltpu.sync_copy(zdeg_hbm, deg_sh)
      pltpu.sync_copy(ones_hbm, ones_v)
    plsc.subcore_barrier()

    def body(g, carry):
      g8 = pl.multiple_of(g * _G, _G)
      pltpu.sync_copy(src_hbm.at[wid, pl.ds(g8, _G)], srcblk)
      pltpu.sync_copy(dst_hbm.at[wid, pl.ds(g8, _G)], dstblk)
      for half in range(_G // _NBUF):
        hs = []
        for k in range(_NBUF):
          j = half * _NBUF + k
          hs.append(pltpu.async_copy(
              table_hbm.at[srcblk.at[j]], rows_v.at[k], sems[k]))
        for k in range(_NBUF):
          j = half * _NBUF + k
          hs[k].wait()
          pltpu.sync_copy(rows_v.at[k], acc_sh.at[dstblk.at[j]], add=True)
          if with_deg:
            pltpu.sync_copy(ones_v, deg_sh.at[dstblk.at[j]], add=True)
      return carry

    lax.fori_loop(0, _NGRP, body, 0)
    plsc.subcore_barrier()

    pltpu.sync_copy(acc_sh.at[pl.ds(s * _RPT, _RPT)],
                    agg_out.at[c, pl.ds(s * _RPT, _RPT)])

    @pl.when(s == _NS - 1)
    def _():
      pltpu.sync_copy(acc_sh.at[pl.ds(_TAIL_OFF, _TAIL)],
                      agg_out.at[c, pl.ds(_TAIL_OFF, _TAIL)])

    if with_deg:
      @pl.when(s == 0)
      def _():
        pltpu.sync_copy(deg_sh, deg_out.at[c])

  return seg


def _prelu(v, a):
  return jnp.where(v >= 0, v, a * v)


_BLK = 1000


def _tc1_body(aggp, degp, x, w0l, b0l, w0r, wskip, a0, hin2, degc):
  agg = aggp[0] + aggp[1]
  deg = jnp.maximum(degp[0] + degp[1], 1.0)
  mean = agg / deg
  h = (jnp.dot(mean, w0l[...], preferred_element_type=jnp.float32)
       + b0l[...]
       + jnp.dot(x[...], w0r[...], preferred_element_type=jnp.float32))
  h = _prelu(_prelu(h, a0[...]), a0[...])
  hin2[...] = h + jnp.dot(x[...], wskip[...], preferred_element_type=jnp.float32)
  degc[...] = deg


def _tc2_body(aggp, degc, hin2, w1l, b1l, w1r, a1, out):
  mean = (aggp[0] + aggp[1]) / degc[...]
  h = (jnp.dot(mean, w1l[...], preferred_element_type=jnp.float32)
       + b1l[...]
       + jnp.dot(hin2[...], w1r[...], preferred_element_type=jnp.float32))
  out[...] = _prelu(h, a1[...])


def _w_spec():
  return pl.BlockSpec((_D, _D), lambda i: (0, 0))


def _v_spec():
  return pl.BlockSpec((1, _D), lambda i: (0, 0))


def _row_spec(d=_D):
  return pl.BlockSpec((_BLK, d), lambda i: (i, 0))


def _part_spec(d=_D):
  return pl.BlockSpec((_NC, _BLK, d), lambda i: (0, i, 0))


def _tc1(aggp, degp, x, w0l, b0l, w0r, wskip, a0):
  return pl.pallas_call(
      _tc1_body,
      grid=(_N // _BLK,),
      in_specs=[_part_spec(), _part_spec(1), _row_spec(), _w_spec(), _v_spec(),
                _w_spec(), _w_spec(), _v_spec()],
      out_specs=[_row_spec(), _row_spec(1)],
      out_shape=[jax.ShapeDtypeStruct((_N, _D), jnp.float32),
                 jax.ShapeDtypeStruct((_N, 1), jnp.float32)],
  )(aggp, degp, x, w0l, b0l, w0r, wskip, a0)


def _tc2(aggp, degc, hin2, w1l, b1l, w1r, a1):
  return pl.pallas_call(
      _tc2_body,
      grid=(_N // _BLK,),
      in_specs=[_part_spec(), _row_spec(1), _row_spec(), _w_spec(), _v_spec(),
                _w_spec(), _v_spec()],
      out_specs=_row_spec(),
      out_shape=jax.ShapeDtypeStruct((_N, _D), jnp.float32),
  )(aggp, degc, hin2, w1l, b1l, w1r, a1)


def kernel(x, edge_index, edge_weight, W0l, b0l, W0r, W1l, b1l, W1r, Wskip, a0, a1):
  del edge_weight
  npad = _EPAD - _E
  src = jnp.concatenate(
      [edge_index[0].astype(jnp.int32),
       jnp.arange(npad, dtype=jnp.int32) % _N])
  dst = jnp.concatenate(
      [edge_index[1].astype(jnp.int32),
       _N + jnp.arange(npad, dtype=jnp.int32) % _NSINK])
  src3 = src.reshape(_NW, _NCHUNK, _C)
  dst3 = dst.reshape(_NW, _NCHUNK, _C)
  ones = jnp.ones((_C,), jnp.float32)
  zrow = jnp.zeros((_N, _D), jnp.float32)
  zdeg = jnp.zeros((_NDEG,), jnp.float32)
  seg1 = _make_seg_sum(True)
  seg2 = _make_seg_sum(False)

  b0 = b0l.reshape(1, _D)
  b1 = b1l.reshape(1, _D)
  a0r = a0.reshape(1, _D)
  a1r = a1.reshape(1, _D)

  agg1p, deg1p = seg1(x, src3, dst3, ones, zrow, zdeg)
  deg1p = deg1p[:, :_N].reshape(_NC, _N, 1)
  hin2, degc = _tc1(agg1p, deg1p, x, W0l, b0, W0r, Wskip, a0r)
  (agg2p,) = seg2(hin2, src3, dst3, ones, zrow, zdeg)
  return _tc2(agg2p, degc, hin2, W1l, b1, W1r, a1r)


# speedup vs baseline: 2.8231x; 1.0518x over previous
import functools

import jax
import jax.numpy as jnp
from jax import lax
from jax.experimental import pallas as pl
from jax.experimental.pallas import tpu as pltpu
from jax.experimental.pallas import tpu_sc as plsc

_N = 10000
_E = 320000
_D = 128
_NC = 2
_NS = 16
_NW = _NC * _NS
_C = 128
_NCHUNK = 80
_PER_W = _NCHUNK * _C
_EPAD = _NW * _PER_W
_NBUF = 2
_G = 8
_NGRP = _NCHUNK // _G
_NSINK = 128
_NDEG = 10240
_RPT = 624
_TAIL = _N - _NS * _RPT
_TAIL_OFF = _NS * _RPT


def _make_seg_sum(with_deg):
  mesh = plsc.VectorSubcoreMesh(core_axis_name="c", subcore_axis_name="s")
  out_type = [jax.ShapeDtypeStruct((_NC, _N, _D), jnp.float32)]
  if with_deg:
    out_type.append(jax.ShapeDtypeStruct((_NC, _NDEG), jnp.float32))

  @functools.partial(
      pl.kernel,
      out_type=out_type,
      mesh=mesh,
      scratch_types=[
          pltpu.VMEM((_G, _C), jnp.int32),
          pltpu.VMEM((_G, _C), jnp.int32),
          pltpu.VMEM((_NBUF, _C, _D), jnp.float32),
          pltpu.VMEM((_C,), jnp.float32),
          pltpu.VMEM_SHARED((_N + _NSINK, _D), jnp.float32),
          pltpu.VMEM_SHARED((_NDEG,), jnp.float32),
          pltpu.SemaphoreType.DMA,
          pltpu.SemaphoreType.DMA,
          pltpu.SemaphoreType.DMA,
          pltpu.SemaphoreType.DMA,
      ],
  )
  def seg(table_hbm, src_hbm, dst_hbm, ones_hbm, zrow_hbm, zdeg_hbm,
          *out_and_scratch):
    if with_deg:
      (agg_out, deg_out, srcblk, dstblk, rows_v, ones_v, acc_sh, deg_sh,
       sem0, sem1, sem2, sem3) = out_and_scratch
    else:
      (agg_out, srcblk, dstblk, rows_v, ones_v, acc_sh, deg_sh,
       sem0, sem1, sem2, sem3) = out_and_scratch
      deg_out = None
    sems = (sem0, sem1, sem2, sem3)
    c = lax.axis_index("c")
    s = lax.axis_index("s")
    wid = s * _NC + c
    pltpu.sync_copy(zrow_hbm.at[pl.ds(s * _RPT, _RPT)],
                    acc_sh.at[pl.ds(s * _RPT, _RPT)])

    @pl.when(s == _NS - 1)
    def _():
      pltpu.sync_copy(zrow_hbm.at[pl.ds(_TAIL_OFF, _TAIL)],
                      acc_sh.at[pl.ds(_TAIL_OFF, _TAIL)])

    if with_deg:
      @pl.when(s == 0)
      def _():
        pltpu.sync_copy(zdeg_hbm, deg_sh)
      pltpu.sync_copy(ones_hbm, ones_v)
    plsc.subcore_barrier()

    def body(g, carry):
      g8 = pl.multiple_of(g * _G, _G)
      pltpu.sync_copy(src_hbm.at[wid, pl.ds(g8, _G)], srcblk)
      pltpu.sync_copy(dst_hbm.at[wid, pl.ds(g8, _G)], dstblk)
      for half in range(_G // _NBUF):
        hs = []
        for k in range(_NBUF):
          j = half * _NBUF + k
          hs.append(pltpu.async_copy(
              table_hbm.at[srcblk.at[j]], rows_v.at[k], sems[k]))
        for k in range(_NBUF):
          j = half * _NBUF + k
          hs[k].wait()
          pltpu.sync_copy(rows_v.at[k], acc_sh.at[dstblk.at[j]], add=True)
          if with_deg:
            pltpu.sync_copy(ones_v, deg_sh.at[dstblk.at[j]], add=True)
      return carry

    lax.fori_loop(0, _NGRP, body, 0)
    plsc.subcore_barrier()

    pltpu.sync_copy(acc_sh.at[pl.ds(s * _RPT, _RPT)],
                    agg_out.at[c, pl.ds(s * _RPT, _RPT)])

    @pl.when(s == _NS - 1)
    def _():
      pltpu.sync_copy(acc_sh.at[pl.ds(_TAIL_OFF, _TAIL)],
                      agg_out.at[c, pl.ds(_TAIL_OFF, _TAIL)])

    if with_deg:
      @pl.when(s == 0)
      def _():
        pltpu.sync_copy(deg_sh, deg_out.at[c])

  return seg


def _prelu(v, a):
  return jnp.where(v >= 0, v, a * v)


_BLK = 1000


def _tc1_body(aggp, degp, x, w0l, b0l, w0r, wskip, a0, hin2, degc):
  agg = aggp[0] + aggp[1]
  deg = jnp.maximum(degp[0] + degp[1], 1.0)
  mean = agg / deg
  h = (jnp.dot(mean, w0l[...], preferred_element_type=jnp.float32)
       + b0l[...]
       + jnp.dot(x[...], w0r[...], preferred_element_type=jnp.float32))
  h = _prelu(_prelu(h, a0[...]), a0[...])
  hin2[...] = h + jnp.dot(x[...], wskip[...], preferred_element_type=jnp.float32)
  degc[...] = deg


def _tc2_body(aggp, degc, hin2, w1l, b1l, w1r, a1, out):
  mean = (aggp[0] + aggp[1]) / degc[...]
  h = (jnp.dot(mean, w1l[...], preferred_element_type=jnp.float32)
       + b1l[...]
       + jnp.dot(hin2[...], w1r[...], preferred_element_type=jnp.float32))
  out[...] = _prelu(h, a1[...])


def _w_spec():
  return pl.BlockSpec((_D, _D), lambda i: (0, 0))


def _v_spec():
  return pl.BlockSpec((1, _D), lambda i: (0, 0))


def _row_spec(d=_D):
  return pl.BlockSpec((_BLK, d), lambda i: (i, 0))


def _part_spec(d=_D):
  return pl.BlockSpec((_NC, _BLK, d), lambda i: (0, i, 0))


def _tc1(aggp, degp, x, w0l, b0l, w0r, wskip, a0):
  return pl.pallas_call(
      _tc1_body,
      grid=(_N // _BLK,),
      in_specs=[_part_spec(), _part_spec(1), _row_spec(), _w_spec(), _v_spec(),
                _w_spec(), _w_spec(), _v_spec()],
      out_specs=[_row_spec(), _row_spec(1)],
      out_shape=[jax.ShapeDtypeStruct((_N, _D), jnp.float32),
                 jax.ShapeDtypeStruct((_N, 1), jnp.float32)],
  )(aggp, degp, x, w0l, b0l, w0r, wskip, a0)


def _tc2(aggp, degc, hin2, w1l, b1l, w1r, a1):
  return pl.pallas_call(
      _tc2_body,
      grid=(_N // _BLK,),
      in_specs=[_part_spec(), _row_spec(1), _row_spec(), _w_spec(), _v_spec(),
                _w_spec(), _v_spec()],
      out_specs=_row_spec(),
      out_shape=jax.ShapeDtypeStruct((_N, _D), jnp.float32),
  )(aggp, degc, hin2, w1l, b1l, w1r, a1)


def kernel(x, edge_index, edge_weight, W0l, b0l, W0r, W1l, b1l, W1r, Wskip, a0, a1):
  del edge_weight
  npad = _EPAD - _E
  src = jnp.concatenate(
      [edge_index[0].astype(jnp.int32),
       jnp.arange(npad, dtype=jnp.int32) % _N])
  dst = jnp.concatenate(
      [edge_index[1].astype(jnp.int32),
       _N + jnp.arange(npad, dtype=jnp.int32) % _NSINK])
  src3 = src.reshape(_NW, _NCHUNK, _C)
  dst3 = dst.reshape(_NW, _NCHUNK, _C)
  ones = jnp.ones((_C,), jnp.float32)
  zrow = jnp.zeros((_N, _D), jnp.float32)
  zdeg = jnp.zeros((_NDEG,), jnp.float32)
  seg1 = _make_seg_sum(True)
  seg2 = _make_seg_sum(False)

  b0 = b0l.reshape(1, _D)
  b1 = b1l.reshape(1, _D)
  a0r = a0.reshape(1, _D)
  a1r = a1.reshape(1, _D)

  agg1p, deg1p = seg1(x, src3, dst3, ones, zrow, zdeg)
  deg1p = deg1p[:, :_N].reshape(_NC, _N, 1)
  hin2, degc = _tc1(agg1p, deg1p, x, W0l, b0, W0r, Wskip, a0r)
  (agg2p,) = seg2(hin2, src3, dst3, ones, zrow, zdeg)
  return _tc2(agg2p, degc, hin2, W1l, b1, W1r, a1r)
